# baseline (speedup 1.0000x reference)
import functools

import jax
import jax.numpy as jnp
from jax import lax
from jax.experimental import pallas as pl
from jax.experimental.pallas import tpu as pltpu
from jax.experimental.pallas import tpu_sc as plsc

N = 10000
NP = 10240
E = 320000
D = 128
H = 64
NSUB = 16
EPT = E // NSUB
CH = 32
NCHUNK = EPT // CH
RPS = NP // NSUB
ZROWS = 32
BE = 2000
NEB = E // BE


def _prep_body(nodes, ehw, ehb, aw, ab, bw, bb, dw, db, ew, eb, eew, eeb, cw,
               cb, h0_o, ah_o, tab_o, cc_o):
    h0 = jnp.dot(nodes[...], ehw[...], preferred_element_type=jnp.float32)
    h0 = h0 + ehb[...]
    h0_o[...] = h0
    ah_o[...] = jnp.dot(h0, aw[...], preferred_element_type=jnp.float32) + ab[...]
    bh = jnp.dot(h0, bw[...], preferred_element_type=jnp.float32) + bb[...]
    dh = jnp.dot(h0, dw[...], preferred_element_type=jnp.float32) + db[...]
    eh = jnp.dot(h0, ew[...], preferred_element_type=jnp.float32) + eb[...]
    c1 = jnp.dot(eew[...], cw[...], preferred_element_type=jnp.float32)
    c2 = jnp.dot(eeb[...], cw[...], preferred_element_type=jnp.float32) + cb[...]
    ehc = eh + c2
    tab_o[0:N, 0:H] = dh[:, 0:H]
    tab_o[0:N, H:D] = bh[:, 0:H]
    tab_o[NP:NP + N, 0:H] = dh[:, H:D]
    tab_o[NP:NP + N, H:D] = bh[:, H:D]
    tab_o[2 * NP:2 * NP + N, 0:H] = ehc[:, 0:H]
    tab_o[3 * NP:3 * NP + N, 0:H] = ehc[:, H:D]
    cc_o[0:1, 0:1, 0:H] = c1[:, None, 0:H]
    cc_o[0:1, 0:1, H:D] = c2[:, None, 0:H]
    cc_o[1:2, 0:1, 0:H] = c1[:, None, H:D]
    cc_o[1:2, 0:1, H:D] = c2[:, None, H:D]


def _tc_prep(nodes_feat, emb_h_w, emb_h_b, aw, ab, bw, bb, dw, db, ew, eb,
             eew, eeb, cw, cb):
    return pl.pallas_call(
        _prep_body,
        out_shape=(
            jax.ShapeDtypeStruct((N, D), jnp.float32),
            jax.ShapeDtypeStruct((N, D), jnp.float32),
            jax.ShapeDtypeStruct((4 * NP, D), jnp.float32),
            jax.ShapeDtypeStruct((2, 1, D), jnp.float32),
        ),
    )(nodes_feat, emb_h_w, emb_h_b, aw, ab, bw, bb, dw, db, ew, eb, eew, eeb,
      cw, cb)


def _make_edge_pass(layer):
    mesh = plsc.VectorSubcoreMesh(core_axis_name="c", subcore_axis_name="s")
    if layer == 0:
        out_type = (
            jax.ShapeDtypeStruct((2 * NP, D), jnp.float32),
            jax.ShapeDtypeStruct((2 * E, H), jnp.float32),
        )
    else:
        out_type = jax.ShapeDtypeStruct((2 * NP, D), jnp.float32)
    scratch = [
        pltpu.VMEM((2, 2 * CH), jnp.int32),
        pltpu.VMEM((2, CH), jnp.int32),
        pltpu.VMEM((2, 2 * CH, D), jnp.float32),
        pltpu.VMEM((2, CH, D), jnp.float32),
        pltpu.VMEM_SHARED((NP, D), jnp.float32),
        pltpu.SemaphoreType.DMA,
        pltpu.SemaphoreType.DMA,
        pltpu.SemaphoreType.DMA,
        pltpu.SemaphoreType.DMA,
        pltpu.SemaphoreType.DMA,
        pltpu.SemaphoreType.DMA,
    ]
    if layer == 0:
        scratch += [
            pltpu.SemaphoreType.DMA,
            pltpu.SemaphoreType.DMA,
            pltpu.VMEM((2, CH, 16), jnp.float32),
            pltpu.VMEM((1, D), jnp.float32),
            pltpu.VMEM((2, CH, H), jnp.float32),
        ]
    else:
        scratch += [
            pltpu.VMEM((2, CH, H), jnp.float32),
        ]

    def body(*refs):
        if layer == 0:
            (tab, cc, rec, frec, zeros,
             nd_o, eij_o,
             vsd, vdsc, gbuf, bnd, spacc,
             si0, si1, sg0, sg1, ss0, ss1, sw0, sw1,
             fvr, cv, beij) = refs
            semw = (sw0, sw1)
        else:
            (tab, celo, cehi, rec, zeros,
             nd_o,
             vsd, vdsc, gbuf, bnd, spacc,
             si0, si1, sg0, sg1, ss0, ss1,
             bce) = refs
        semi = (si0, si1)
        semg = (sg0, sg1)
        sems = (ss0, ss1)
        c = lax.axis_index("c")
        s = lax.axis_index("s")
        cn = c * NP

        pltpu.sync_copy(zeros, spacc.at[pl.ds(s * RPS, RPS)])
        if layer == 0:
            pltpu.sync_copy(cc.at[c], cv)
        plsc.subcore_barrier()

        gbase = s * NCHUNK
        ebase = s * EPT

        def issue_idx(kt, b):
            g = gbase + jnp.minimum(kt, NCHUNK - 1)
            pltpu.async_copy(rec.at[g], vsd.at[b], semi[b])
            if layer == 0:
                pltpu.async_copy(frec.at[g], fvr.at[b], semi[b])
            else:
                base = ebase + jnp.minimum(kt, NCHUNK - 1) * CH

                @pl.when(c == 0)
                def _():
                    pltpu.async_copy(celo.at[pl.ds(base, CH)], bce.at[b],
                                     semi[b])

                @pl.when(c == 1)
                def _():
                    pltpu.async_copy(cehi.at[pl.ds(base, CH)], bce.at[b],
                                     semi[b])

        def wait_idx(b):
            pltpu.make_async_copy(rec.at[0], vsd.at[b], semi[b]).wait()
            if layer == 0:
                pltpu.make_async_copy(frec.at[0], fvr.at[b], semi[b]).wait()
            else:
                pltpu.make_async_copy(celo.at[pl.ds(0, CH)], bce.at[b],
                                      semi[b]).wait()

        def adjust(b):
            for i in range(CH // 16):
                sl = pl.ds(i * 16, 16)
                sl2 = pl.ds(CH + i * 16, 16)
                vsd[b, sl] = vsd[b, sl] + cn
                vsd[b, sl2] = vsd[b, sl2] + (cn + 2 * NP)

        def issue_gather(b):
            pltpu.async_copy(tab.at[vsd.at[b]], gbuf.at[b], semg[b])

        def wait_gather(b):
            pltpu.make_async_copy(tab.at[pl.ds(0, 2 * CH)], gbuf.at[b],
                                  semg[b]).wait()

        def issue_out(kt, b):
            pltpu.async_copy(bnd.at[b], spacc.at[vdsc.at[b]], sems[b],
                             add=True)
            if layer == 0:
                base = ebase + kt * CH
                pltpu.async_copy(beij.at[b], eij_o.at[pl.ds(c * E + base, CH)],
                                 semw[b])

        def wait_out(b):
            pltpu.make_async_copy(tab.at[pl.ds(0, CH)], bnd.at[b],
                                  sems[b]).wait()
            if layer == 0:
                pltpu.make_async_copy(beij.at[b],
                                      eij_o.at[pl.ds(c * E, CH)],
                                      semw[b]).wait()

        def compute(b):
            for i in range(CH // 16):
                sl = pl.ds(i * 16, 16)
                vdsc[b, sl] = vsd[b, pl.ds(CH + i * 16, 16)] - (cn + 2 * NP)

            def edge(j):
                if layer == 0:
                    fe = fvr[b, j, :]
                for d in range(4):
                    sl = pl.ds(d * 16, 16)
                    sl2 = pl.ds(H + d * 16, 16)
                    x = gbuf[b, j, sl] + gbuf[b, CH + j, sl]
                    if layer == 0:
                        x = x + fe * cv[0, sl]
                    else:
                        x = x + bce[b, j, sl]
                    sig = 1.0 / (1.0 + jnp.exp(-x))
                    bnd[b, j, sl] = sig * gbuf[b, j, sl2]
                    bnd[b, j, sl2] = sig
                    if layer == 0:
                        beij[b, j, sl] = x
            plsc.parallel_loop(0, CH, 1, unroll=2)(edge)

        issue_idx(0, 0)
        issue_idx(1, 1)
        wait_idx(0)
        adjust(0)
        issue_gather(0)

        def step(kk, p_or_none, b, tail):
            nb = 1 - b
            wait_idx(nb)
            adjust(nb)
            wait_gather(b)
            if not tail:
                issue_gather(nb)
            if tail:
                wait_out(b)
            else:
                @pl.when(p_or_none >= 1)
                def _():
                    wait_out(b)
            compute(b)
            issue_out(kk, b)
            if not tail:
                issue_idx(kk + 2, b)

        def pair(p, carry):
            for b in (0, 1):
                step(2 * p + b, p, b, False)
            return carry
        lax.fori_loop(0, NCHUNK // 2, pair, 0)
        step(NCHUNK - 1, None, 0, True)
        wait_out(1)
        wait_out(0)

        plsc.subcore_barrier()
        pltpu.sync_copy(spacc.at[pl.ds(s * RPS, RPS)],
                        nd_o.at[pl.ds(cn + s * RPS, RPS)])

    return pl.kernel(body, out_type=out_type, mesh=mesh,
                     scratch_types=scratch)


def _mid_node_body(nd, ah1, h0, ssum, ssq, sn, se, bhg, bhb, beg, beb,
                   h1_o, scl_o, shf_o):
    num = jnp.concatenate([nd[0:N, 0:H], nd[NP:NP + N, 0:H]], axis=1)
    den = jnp.concatenate([nd[0:N, H:D], nd[NP:NP + N, H:D]], axis=1)
    hn = (ah1[...] + num / (den + 1e-6)) * sn[...]
    mu = jnp.mean(hn, axis=0, keepdims=True)
    var = jnp.mean((hn - mu) * (hn - mu), axis=0, keepdims=True)
    hn = bhg[...] * (hn - mu) / jnp.sqrt(var + 1e-5) + bhb[...]
    h1_o[...] = h0[...] + jnp.maximum(hn, 0.0)
    sev = se[...]
    mue = sev * ssum[...] / E
    vare = sev * sev * ssq[...] / E - mue * mue
    g_over = beg[...] / jnp.sqrt(vare + 1e-5)
    scl_o[...] = sev * g_over
    shf_o[...] = beb[...] - mue * g_over


def _mid_tab_body(h1, aw, ab, bw, bb, dw, db, ew, eb, ah2_o, tab_o):
    h1v = h1[...]
    ah2_o[...] = jnp.dot(h1v, aw[...], preferred_element_type=jnp.float32) + ab[...]
    bh = jnp.dot(h1v, bw[...], preferred_element_type=jnp.float32) + bb[...]
    dh = jnp.dot(h1v, dw[...], preferred_element_type=jnp.float32) + db[...]
    eh = jnp.dot(h1v, ew[...], preferred_element_type=jnp.float32) + eb[...]
    tab_o[0:N, 0:H] = dh[:, 0:H]
    tab_o[0:N, H:D] = bh[:, 0:H]
    tab_o[NP:NP + N, 0:H] = dh[:, H:D]
    tab_o[NP:NP + N, H:D] = bh[:, H:D]
    tab_o[2 * NP:2 * NP + N, 0:H] = eh[:, 0:H]
    tab_o[3 * NP:3 * NP + N, 0:H] = eh[:, H:D]


def _tc_mid(nd, ah1, h0, ssum, ssq, sn, se, bhg, bhb, beg, beb, aw, ab, bw,
            bb, dw, db, ew, eb):
    h1, scl, shf = pl.pallas_call(
        _mid_node_body,
        out_shape=(
            jax.ShapeDtypeStruct((N, D), jnp.float32),
            jax.ShapeDtypeStruct((1, D), jnp.float32),
            jax.ShapeDtypeStruct((1, D), jnp.float32),
        ),
    )(nd, ah1, h0, ssum, ssq, sn, se, bhg, bhb, beg, beb)
    ah2, tab = pl.pallas_call(
        _mid_tab_body,
        out_shape=(
            jax.ShapeDtypeStruct((N, D), jnp.float32),
            jax.ShapeDtypeStruct((4 * NP, D), jnp.float32),
        ),
    )(h1, aw, ab, bw, bb, dw, db, ew, eb)
    return h1, ah2, tab, scl, shf


def _estat_body(lo, hi, ssum_o, ssq_o):
    eij = jnp.concatenate([lo[...], hi[...]], axis=1)
    bs = jnp.sum(eij, axis=0, keepdims=True)
    bq = jnp.sum(eij * eij, axis=0, keepdims=True)

    @pl.when(pl.program_id(0) == 0)
    def _():
        ssum_o[...] = jnp.zeros_like(ssum_o)
        ssq_o[...] = jnp.zeros_like(ssq_o)
    ssum_o[...] += bs
    ssq_o[...] += bq


def _tc_estat(eij):
    full = lambda i: (0, 0)
    return pl.pallas_call(
        _estat_body,
        grid=(NEB,),
        in_specs=[
            pl.BlockSpec((BE, H), lambda i: (i, 0)),
            pl.BlockSpec((BE, H), lambda i: (i + NEB, 0)),
        ],
        out_specs=[
            pl.BlockSpec((1, D), full),
            pl.BlockSpec((1, D), full),
        ],
        out_shape=(
            jax.ShapeDtypeStruct((1, D), jnp.float32),
            jax.ShapeDtypeStruct((1, D), jnp.float32),
        ),
    )(eij, eij)


def _ce_body(lo, hi, f, scl, shf, we, be, cw, cb, celo_o, cehi_o):
    eij = jnp.concatenate([lo[...], hi[...]], axis=1)
    en = jnp.maximum(eij * scl[...] + shf[...], 0.0)
    e1 = f[...] * we[...] + be[...] + en
    ce = jnp.dot(e1, cw[...], preferred_element_type=jnp.float32) + cb[...]
    celo_o[...] = ce[:, 0:H]
    cehi_o[...] = ce[:, H:D]


def _tc_ce(eij, f, scl, shf, we, be, cw, cb):
    full = lambda i: (0, 0)
    return pl.pallas_call(
        _ce_body,
        grid=(NEB,),
        in_specs=[
            pl.BlockSpec((BE, H), lambda i: (i, 0)),
            pl.BlockSpec((BE, H), lambda i: (i + NEB, 0)),
            pl.BlockSpec((BE, 1), lambda i: (i, 0)),
            pl.BlockSpec((1, D), full),
            pl.BlockSpec((1, D), full),
            pl.BlockSpec((1, D), full),
            pl.BlockSpec((1, D), full),
            pl.BlockSpec((D, D), full),
            pl.BlockSpec((1, D), full),
        ],
        out_specs=[
            pl.BlockSpec((BE, H), lambda i: (i, 0)),
            pl.BlockSpec((BE, H), lambda i: (i, 0)),
        ],
        out_shape=(
            jax.ShapeDtypeStruct((E, H), jnp.float32),
            jax.ShapeDtypeStruct((E, H), jnp.float32),
        ),
    )(eij, eij, f, scl, shf, we, be, cw, cb)


def _post_body(nd, ah2, h1, sn, bhg, bhb, row, out):
    num = jnp.concatenate([nd[0:N, 0:H], nd[NP:NP + N, 0:H]], axis=1)
    den = jnp.concatenate([nd[0:N, H:D], nd[NP:NP + N, H:D]], axis=1)
    hn = (ah2[...] + num / (den + 1e-6)) * sn[...]
    mu = jnp.mean(hn, axis=0, keepdims=True)
    var = jnp.mean((hn - mu) * (hn - mu), axis=0, keepdims=True)
    hn = bhg[...] * (hn - mu) / jnp.sqrt(var + 1e-5) + bhb[...]
    h2 = h1[...] + jnp.maximum(hn, 0.0)
    hg = jnp.mean(h2, axis=0, keepdims=True)
    out[...] = jnp.dot(hg, row[...], preferred_element_type=jnp.float32)


def _tc_post(nd, ah2, h1, sn, bhg, bhb, row):
    nc = row.shape[1]
    return pl.pallas_call(
        _post_body,
        out_shape=jax.ShapeDtypeStruct((1, nc), jnp.float32),
    )(nd, ah2, h1, sn, bhg, bhb, row)


def kernel(nodes_feat, edge_index, edges_feat, snorm_n, snorm_e, emb_h_w,
           emb_h_b, emb_e_w, emb_e_b, A_w, A_b, B_w, B_b, C_w, C_b, D_w, D_b,
           E_w, E_b, bn_h_g, bn_h_b, bn_e_g, bn_e_b, ro_w):
    src = edge_index[0]
    dst = edge_index[1]
    f1 = edges_feat[:, 0]
    sn = snorm_n[0:1, 0:1]
    se = snorm_e[0:1, 0:1]
    r1 = lambda v: v.reshape(1, D)
    rec = jnp.concatenate(
        [src.reshape(E // CH, CH), dst.reshape(E // CH, CH)], axis=1)
    frec = jnp.broadcast_to(f1[:, None], (E, 16)).reshape(E // CH, CH, 16)
    zeros = jnp.zeros((RPS, D), jnp.float32)

    h0, ah1, tab1, cc = _tc_prep(
        nodes_feat, emb_h_w, r1(emb_h_b), A_w[0], r1(A_b[0]), B_w[0],
        r1(B_b[0]), D_w[0], r1(D_b[0]), E_w[0], r1(E_b[0]), emb_e_w,
        r1(emb_e_b), C_w[0], r1(C_b[0]))

    nd1, eij1 = _make_edge_pass(0)(tab1, cc, rec, frec, zeros)
    ssum, ssq = _tc_estat(eij1)

    h1, ah2, tab2, scl, shf = _tc_mid(
        nd1, ah1, h0, ssum, ssq, sn, se, r1(bn_h_g[0]), r1(bn_h_b[0]),
        r1(bn_e_g[0]), r1(bn_e_b[0]), A_w[1], r1(A_b[1]), B_w[1], r1(B_b[1]),
        D_w[1], r1(D_b[1]), E_w[1], r1(E_b[1]))

    celo, cehi = _tc_ce(eij1, edges_feat, scl, shf, emb_e_w, r1(emb_e_b),
                        C_w[1], r1(C_b[1]))

    nd2 = _make_edge_pass(1)(tab2, celo, cehi, rec, zeros)

    return _tc_post(nd2, ah2, h1, sn, r1(bn_h_g[1]), r1(bn_h_b[1]), ro_w)


# speedup vs baseline: 1.0030x; 1.0030x over previous
import functools

import jax
import jax.numpy as jnp
from jax import lax
from jax.experimental import pallas as pl
from jax.experimental.pallas import tpu as pltpu
from jax.experimental.pallas import tpu_sc as plsc

N = 10000
NP = 10240
E = 320000
D = 128
H = 64
NSUB = 16
EPT = E // NSUB
CH = 32
NCHUNK = EPT // CH
RPS = NP // NSUB
ZROWS = 32
BE = 2000
NEB = E // BE


def _prep_body(nodes, ehw, ehb, aw, ab, bw, bb, dw, db, ew, eb, eew, eeb, cw,
               cb, h0_o, ah_o, tab_o, cc_o):
    h0 = jnp.dot(nodes[...], ehw[...], preferred_element_type=jnp.float32)
    h0 = h0 + ehb[...]
    h0_o[...] = h0
    ah_o[...] = jnp.dot(h0, aw[...], preferred_element_type=jnp.float32) + ab[...]
    bh = jnp.dot(h0, bw[...], preferred_element_type=jnp.float32) + bb[...]
    dh = jnp.dot(h0, dw[...], preferred_element_type=jnp.float32) + db[...]
    eh = jnp.dot(h0, ew[...], preferred_element_type=jnp.float32) + eb[...]
    c1 = jnp.dot(eew[...], cw[...], preferred_element_type=jnp.float32)
    c2 = jnp.dot(eeb[...], cw[...], preferred_element_type=jnp.float32) + cb[...]
    ehc = eh + c2
    tab_o[0:N, 0:H] = dh[:, 0:H]
    tab_o[0:N, H:D] = bh[:, 0:H]
    tab_o[NP:NP + N, 0:H] = dh[:, H:D]
    tab_o[NP:NP + N, H:D] = bh[:, H:D]
    tab_o[2 * NP:2 * NP + N, 0:H] = ehc[:, 0:H]
    tab_o[3 * NP:3 * NP + N, 0:H] = ehc[:, H:D]
    cc_o[0:1, 0:1, 0:H] = c1[:, None, 0:H]
    cc_o[0:1, 0:1, H:D] = c2[:, None, 0:H]
    cc_o[1:2, 0:1, 0:H] = c1[:, None, H:D]
    cc_o[1:2, 0:1, H:D] = c2[:, None, H:D]


def _tc_prep(nodes_feat, emb_h_w, emb_h_b, aw, ab, bw, bb, dw, db, ew, eb,
             eew, eeb, cw, cb):
    return pl.pallas_call(
        _prep_body,
        out_shape=(
            jax.ShapeDtypeStruct((N, D), jnp.float32),
            jax.ShapeDtypeStruct((N, D), jnp.float32),
            jax.ShapeDtypeStruct((4 * NP, D), jnp.float32),
            jax.ShapeDtypeStruct((2, 1, D), jnp.float32),
        ),
    )(nodes_feat, emb_h_w, emb_h_b, aw, ab, bw, bb, dw, db, ew, eb, eew, eeb,
      cw, cb)


def _make_edge_pass(layer):
    mesh = plsc.VectorSubcoreMesh(core_axis_name="c", subcore_axis_name="s")
    if layer == 0:
        out_type = (
            jax.ShapeDtypeStruct((2 * NP, D), jnp.float32),
            jax.ShapeDtypeStruct((2 * E, H), jnp.float32),
        )
    else:
        out_type = jax.ShapeDtypeStruct((2 * NP, D), jnp.float32)
    scratch = [
        pltpu.VMEM((2, 2 * CH), jnp.int32),
        pltpu.VMEM((2, CH), jnp.int32),
        pltpu.VMEM((2, 2 * CH, D), jnp.float32),
        pltpu.VMEM((2, CH, D), jnp.float32),
        pltpu.VMEM_SHARED((NP, D), jnp.float32),
        pltpu.SemaphoreType.DMA,
        pltpu.SemaphoreType.DMA,
        pltpu.SemaphoreType.DMA,
        pltpu.SemaphoreType.DMA,
        pltpu.SemaphoreType.DMA,
        pltpu.SemaphoreType.DMA,
    ]
    if layer == 0:
        scratch += [
            pltpu.SemaphoreType.DMA,
            pltpu.SemaphoreType.DMA,
            pltpu.VMEM((2, CH, 16), jnp.float32),
            pltpu.VMEM((1, D), jnp.float32),
            pltpu.VMEM((2, CH, H), jnp.float32),
        ]
    else:
        scratch += [
            pltpu.VMEM((2, CH, H), jnp.float32),
        ]

    def body(*refs):
        if layer == 0:
            (tab, cc, rec, frec, zeros,
             nd_o, eij_o,
             vsd, vdsc, gbuf, bnd, spacc,
             si0, si1, sg0, sg1, ss0, ss1, sw0, sw1,
             fvr, cv, beij) = refs
            semw = (sw0, sw1)
        else:
            (tab, celo, cehi, rec, zeros,
             nd_o,
             vsd, vdsc, gbuf, bnd, spacc,
             si0, si1, sg0, sg1, ss0, ss1,
             bce) = refs
        semi = (si0, si1)
        semg = (sg0, sg1)
        sems = (ss0, ss1)
        c = lax.axis_index("c")
        s = lax.axis_index("s")
        cn = c * NP

        pltpu.sync_copy(zeros, spacc.at[pl.ds(s * RPS, RPS)])
        if layer == 0:
            pltpu.sync_copy(cc.at[c], cv)
        plsc.subcore_barrier()

        gbase = s * NCHUNK
        ebase = s * EPT

        def issue_idx(kt, b):
            g = gbase + jnp.minimum(kt, NCHUNK - 1)
            pltpu.async_copy(rec.at[g], vsd.at[b], semi[b])
            if layer == 0:
                pltpu.async_copy(frec.at[g], fvr.at[b], semi[b])
            else:
                base = ebase + jnp.minimum(kt, NCHUNK - 1) * CH

                @pl.when(c == 0)
                def _():
                    pltpu.async_copy(celo.at[pl.ds(base, CH)], bce.at[b],
                                     semi[b])

                @pl.when(c == 1)
                def _():
                    pltpu.async_copy(cehi.at[pl.ds(base, CH)], bce.at[b],
                                     semi[b])

        def wait_idx(b):
            pltpu.make_async_copy(rec.at[0], vsd.at[b], semi[b]).wait()
            if layer == 0:
                pltpu.make_async_copy(frec.at[0], fvr.at[b], semi[b]).wait()
            else:
                pltpu.make_async_copy(celo.at[pl.ds(0, CH)], bce.at[b],
                                      semi[b]).wait()

        def adjust(b):
            for i in range(CH // 16):
                sl = pl.ds(i * 16, 16)
                sl2 = pl.ds(CH + i * 16, 16)
                vsd[b, sl] = vsd[b, sl] + cn
                vsd[b, sl2] = vsd[b, sl2] + (cn + 2 * NP)

        def issue_gather(b):
            pltpu.async_copy(tab.at[vsd.at[b]], gbuf.at[b], semg[b])

        def wait_gather(b):
            pltpu.make_async_copy(tab.at[pl.ds(0, 2 * CH)], gbuf.at[b],
                                  semg[b]).wait()

        def issue_out(kt, b):
            pltpu.async_copy(bnd.at[b], spacc.at[vdsc.at[b]], sems[b],
                             add=True)
            if layer == 0:
                base = ebase + kt * CH
                pltpu.async_copy(beij.at[b], eij_o.at[pl.ds(c * E + base, CH)],
                                 semw[b])

        def wait_out(b):
            pltpu.make_async_copy(tab.at[pl.ds(0, CH)], bnd.at[b],
                                  sems[b]).wait()
            if layer == 0:
                pltpu.make_async_copy(beij.at[b],
                                      eij_o.at[pl.ds(c * E, CH)],
                                      semw[b]).wait()

        def compute(b):
            for i in range(CH // 16):
                sl = pl.ds(i * 16, 16)
                vdsc[b, sl] = vsd[b, pl.ds(CH + i * 16, 16)] - (cn + 2 * NP)

            def edge(j):
                if layer == 0:
                    fe = fvr[b, j, :]
                for d in range(4):
                    sl = pl.ds(d * 16, 16)
                    sl2 = pl.ds(H + d * 16, 16)
                    x = gbuf[b, j, sl] + gbuf[b, CH + j, sl]
                    if layer == 0:
                        x = x + fe * cv[0, sl]
                    else:
                        x = x + bce[b, j, sl]
                    sig = 1.0 / (1.0 + jnp.exp(-x))
                    bnd[b, j, sl] = sig * gbuf[b, j, sl2]
                    bnd[b, j, sl2] = sig
                    if layer == 0:
                        beij[b, j, sl] = x
            plsc.parallel_loop(0, CH, 1, unroll=4)(edge)

        issue_idx(0, 0)
        issue_idx(1, 1)
        wait_idx(0)
        adjust(0)
        issue_gather(0)

        def step(kk, p_or_none, b, tail):
            nb = 1 - b
            wait_idx(nb)
            adjust(nb)
            wait_gather(b)
            if not tail:
                issue_gather(nb)
            if tail:
                wait_out(b)
            else:
                @pl.when(p_or_none >= 1)
                def _():
                    wait_out(b)
            compute(b)
            issue_out(kk, b)
            if not tail:
                issue_idx(kk + 2, b)

        def pair(p, carry):
            for b in (0, 1):
                step(2 * p + b, p, b, False)
            return carry
        lax.fori_loop(0, NCHUNK // 2, pair, 0)
        step(NCHUNK - 1, None, 0, True)
        wait_out(1)
        wait_out(0)

        plsc.subcore_barrier()
        pltpu.sync_copy(spacc.at[pl.ds(s * RPS, RPS)],
                        nd_o.at[pl.ds(cn + s * RPS, RPS)])

    return pl.kernel(body, out_type=out_type, mesh=mesh,
                     scratch_types=scratch)


def _mid_node_body(nd, ah1, h0, ssum, ssq, sn, se, bhg, bhb, beg, beb,
                   h1_o, scl_o, shf_o):
    num = jnp.concatenate([nd[0:N, 0:H], nd[NP:NP + N, 0:H]], axis=1)
    den = jnp.concatenate([nd[0:N, H:D], nd[NP:NP + N, H:D]], axis=1)
    hn = (ah1[...] + num / (den + 1e-6)) * sn[...]
    mu = jnp.mean(hn, axis=0, keepdims=True)
    var = jnp.mean((hn - mu) * (hn - mu), axis=0, keepdims=True)
    hn = bhg[...] * (hn - mu) / jnp.sqrt(var + 1e-5) + bhb[...]
    h1_o[...] = h0[...] + jnp.maximum(hn, 0.0)
    sev = se[...]
    mue = sev * ssum[...] / E
    vare = sev * sev * ssq[...] / E - mue * mue
    g_over = beg[...] / jnp.sqrt(vare + 1e-5)
    scl_o[...] = sev * g_over
    shf_o[...] = beb[...] - mue * g_over


def _mid_tab_body(h1, aw, ab, bw, bb, dw, db, ew, eb, ah2_o, tab_o):
    h1v = h1[...]
    ah2_o[...] = jnp.dot(h1v, aw[...], preferred_element_type=jnp.float32) + ab[...]
    bh = jnp.dot(h1v, bw[...], preferred_element_type=jnp.float32) + bb[...]
    dh = jnp.dot(h1v, dw[...], preferred_element_type=jnp.float32) + db[...]
    eh = jnp.dot(h1v, ew[...], preferred_element_type=jnp.float32) + eb[...]
    tab_o[0:N, 0:H] = dh[:, 0:H]
    tab_o[0:N, H:D] = bh[:, 0:H]
    tab_o[NP:NP + N, 0:H] = dh[:, H:D]
    tab_o[NP:NP + N, H:D] = bh[:, H:D]
    tab_o[2 * NP:2 * NP + N, 0:H] = eh[:, 0:H]
    tab_o[3 * NP:3 * NP + N, 0:H] = eh[:, H:D]


def _tc_mid(nd, ah1, h0, ssum, ssq, sn, se, bhg, bhb, beg, beb, aw, ab, bw,
            bb, dw, db, ew, eb):
    h1, scl, shf = pl.pallas_call(
        _mid_node_body,
        out_shape=(
            jax.ShapeDtypeStruct((N, D), jnp.float32),
            jax.ShapeDtypeStruct((1, D), jnp.float32),
            jax.ShapeDtypeStruct((1, D), jnp.float32),
        ),
    )(nd, ah1, h0, ssum, ssq, sn, se, bhg, bhb, beg, beb)
    ah2, tab = pl.pallas_call(
        _mid_tab_body,
        out_shape=(
            jax.ShapeDtypeStruct((N, D), jnp.float32),
            jax.ShapeDtypeStruct((4 * NP, D), jnp.float32),
        ),
    )(h1, aw, ab, bw, bb, dw, db, ew, eb)
    return h1, ah2, tab, scl, shf


def _estat_body(lo, hi, ssum_o, ssq_o):
    eij = jnp.concatenate([lo[...], hi[...]], axis=1)
    bs = jnp.sum(eij, axis=0, keepdims=True)
    bq = jnp.sum(eij * eij, axis=0, keepdims=True)

    @pl.when(pl.program_id(0) == 0)
    def _():
        ssum_o[...] = jnp.zeros_like(ssum_o)
        ssq_o[...] = jnp.zeros_like(ssq_o)
    ssum_o[...] += bs
    ssq_o[...] += bq


def _tc_estat(eij):
    full = lambda i: (0, 0)
    return pl.pallas_call(
        _estat_body,
        grid=(NEB,),
        in_specs=[
            pl.BlockSpec((BE, H), lambda i: (i, 0)),
            pl.BlockSpec((BE, H), lambda i: (i + NEB, 0)),
        ],
        out_specs=[
            pl.BlockSpec((1, D), full),
            pl.BlockSpec((1, D), full),
        ],
        out_shape=(
            jax.ShapeDtypeStruct((1, D), jnp.float32),
            jax.ShapeDtypeStruct((1, D), jnp.float32),
        ),
    )(eij, eij)


def _ce_body(lo, hi, f, scl, shf, we, be, cw, cb, celo_o, cehi_o):
    eij = jnp.concatenate([lo[...], hi[...]], axis=1)
    en = jnp.maximum(eij * scl[...] + shf[...], 0.0)
    e1 = f[...] * we[...] + be[...] + en
    ce = jnp.dot(e1, cw[...], preferred_element_type=jnp.float32) + cb[...]
    celo_o[...] = ce[:, 0:H]
    cehi_o[...] = ce[:, H:D]


def _tc_ce(eij, f, scl, shf, we, be, cw, cb):
    full = lambda i: (0, 0)
    return pl.pallas_call(
        _ce_body,
        grid=(NEB,),
        in_specs=[
            pl.BlockSpec((BE, H), lambda i: (i, 0)),
            pl.BlockSpec((BE, H), lambda i: (i + NEB, 0)),
            pl.BlockSpec((BE, 1), lambda i: (i, 0)),
            pl.BlockSpec((1, D), full),
            pl.BlockSpec((1, D), full),
            pl.BlockSpec((1, D), full),
            pl.BlockSpec((1, D), full),
            pl.BlockSpec((D, D), full),
            pl.BlockSpec((1, D), full),
        ],
        out_specs=[
            pl.BlockSpec((BE, H), lambda i: (i, 0)),
            pl.BlockSpec((BE, H), lambda i: (i, 0)),
        ],
        out_shape=(
            jax.ShapeDtypeStruct((E, H), jnp.float32),
            jax.ShapeDtypeStruct((E, H), jnp.float32),
        ),
    )(eij, eij, f, scl, shf, we, be, cw, cb)


def _post_body(nd, ah2, h1, sn, bhg, bhb, row, out):
    num = jnp.concatenate([nd[0:N, 0:H], nd[NP:NP + N, 0:H]], axis=1)
    den = jnp.concatenate([nd[0:N, H:D], nd[NP:NP + N, H:D]], axis=1)
    hn = (ah2[...] + num / (den + 1e-6)) * sn[...]
    mu = jnp.mean(hn, axis=0, keepdims=True)
    var = jnp.mean((hn - mu) * (hn - mu), axis=0, keepdims=True)
    hn = bhg[...] * (hn - mu) / jnp.sqrt(var + 1e-5) + bhb[...]
    h2 = h1[...] + jnp.maximum(hn, 0.0)
    hg = jnp.mean(h2, axis=0, keepdims=True)
    out[...] = jnp.dot(hg, row[...], preferred_element_type=jnp.float32)


def _tc_post(nd, ah2, h1, sn, bhg, bhb, row):
    nc = row.shape[1]
    return pl.pallas_call(
        _post_body,
        out_shape=jax.ShapeDtypeStruct((1, nc), jnp.float32),
    )(nd, ah2, h1, sn, bhg, bhb, row)


def kernel(nodes_feat, edge_index, edges_feat, snorm_n, snorm_e, emb_h_w,
           emb_h_b, emb_e_w, emb_e_b, A_w, A_b, B_w, B_b, C_w, C_b, D_w, D_b,
           E_w, E_b, bn_h_g, bn_h_b, bn_e_g, bn_e_b, ro_w):
    src = edge_index[0]
    dst = edge_index[1]
    f1 = edges_feat[:, 0]
    sn = snorm_n[0:1, 0:1]
    se = snorm_e[0:1, 0:1]
    r1 = lambda v: v.reshape(1, D)
    rec = jnp.concatenate(
        [src.reshape(E // CH, CH), dst.reshape(E // CH, CH)], axis=1)
    frec = jnp.broadcast_to(f1[:, None], (E, 16)).reshape(E // CH, CH, 16)
    zeros = jnp.zeros((RPS, D), jnp.float32)

    h0, ah1, tab1, cc = _tc_prep(
        nodes_feat, emb_h_w, r1(emb_h_b), A_w[0], r1(A_b[0]), B_w[0],
        r1(B_b[0]), D_w[0], r1(D_b[0]), E_w[0], r1(E_b[0]), emb_e_w,
        r1(emb_e_b), C_w[0], r1(C_b[0]))

    nd1, eij1 = _make_edge_pass(0)(tab1, cc, rec, frec, zeros)
    ssum, ssq = _tc_estat(eij1)

    h1, ah2, tab2, scl, shf = _tc_mid(
        nd1, ah1, h0, ssum, ssq, sn, se, r1(bn_h_g[0]), r1(bn_h_b[0]),
        r1(bn_e_g[0]), r1(bn_e_b[0]), A_w[1], r1(A_b[1]), B_w[1], r1(B_b[1]),
        D_w[1], r1(D_b[1]), E_w[1], r1(E_b[1]))

    celo, cehi = _tc_ce(eij1, edges_feat, scl, shf, emb_e_w, r1(emb_e_b),
                        C_w[1], r1(C_b[1]))

    nd2 = _make_edge_pass(1)(tab2, celo, cehi, rec, zeros)

    return _tc_post(nd2, ah2, h1, sn, r1(bn_h_g[1]), r1(bn_h_b[1]), ro_w)
